# baseline (speedup 1.0000x reference)
import jax
import jax.numpy as jnp
from jax.experimental import pallas as pl
from jax.experimental.pallas import tpu as pltpu

_NUM_OUT = 1000
_BLOCK_B = 512


def _fused_mlp_kernel(x_ref, w1_ref, w2_ref, b2_ref, ot_ref):
    h = jnp.dot(x_ref[...], w1_ref[...], preferred_element_type=jnp.float32)
    h = jnp.maximum(h, 0.0)
    out = jnp.dot(h, w2_ref[...], preferred_element_type=jnp.float32)
    out = out + b2_ref[...]
    ot_ref[...] = out.T[:_NUM_OUT].astype(ot_ref.dtype)


@jax.jit
def kernel(x, w1_p, w2_p, b2_p):
    B, Din = x.shape
    Hp = w1_p.shape[1]
    Cp = w2_p.shape[1]
    bm = _BLOCK_B if B % _BLOCK_B == 0 else B
    out_t = pl.pallas_call(
        _fused_mlp_kernel,
        out_shape=jax.ShapeDtypeStruct((_NUM_OUT, B), x.dtype),
        grid=(B // bm,),
        in_specs=[
            pl.BlockSpec((bm, Din), lambda i: (i, 0)),
            pl.BlockSpec((Din, Hp), lambda i: (0, 0)),
            pl.BlockSpec((Hp, Cp), lambda i: (0, 0)),
            pl.BlockSpec((1, Cp), lambda i: (0, 0)),
        ],
        out_specs=pl.BlockSpec((_NUM_OUT, bm), lambda i: (0, i)),
        compiler_params=pltpu.CompilerParams(
            dimension_semantics=("arbitrary",)),
    )(x, w1_p, w2_p, b2_p)
    return out_t.T
